# Initial kernel scaffold; baseline (speedup 1.0000x reference)
import jax
import jax.numpy as jnp
from jax.experimental import pallas as pl


def kernel(x, edge_index, W1, b1, g1, bb1, W2, b2, g2, bb2, Wd, bd, gf, bf, Wp, bp):
    raise NotImplementedError("write your pallas kernel here")


# speedup vs baseline: 1.0254x; 1.0254x over previous
import jax
import jax.numpy as jnp
from jax.experimental import pallas as pl
from jax.experimental.pallas import tpu as pltpu

N = 10000
E = 320000
D = 128
H = 128


def _dense_bn_body(h_ref, w_ref, b_ref, g_ref, bb_ref, o_ref):
    h = h_ref[...]
    a = jnp.maximum(
        jnp.dot(h, w_ref[...], preferred_element_type=jnp.float32) + b_ref[...],
        0.0,
    )
    mu = jnp.mean(a, axis=0, keepdims=True)
    var = jnp.mean((a - mu) ** 2, axis=0, keepdims=True)
    o_ref[...] = (a - mu) * jax.lax.rsqrt(var + 1e-5) * g_ref[...] + bb_ref[...]


def _dense_bn(h, W, b, g, bb):
    return pl.pallas_call(
        _dense_bn_body,
        out_shape=jax.ShapeDtypeStruct((N, H), jnp.float32),
    )(h, W, b.reshape(1, H), g.reshape(1, H), bb.reshape(1, H))


def _mean_agg(h, src, dst, deg):
    s = jax.ops.segment_sum(h[src], dst, num_segments=N)
    mean = s / jnp.maximum(deg, 1.0)[:, None]
    return jnp.where((deg > 0)[:, None], mean, h)


def _max_agg(h, src, dst, deg):
    mx = jax.ops.segment_max(h[src], dst, num_segments=N)
    return jnp.where((deg > 0)[:, None], mx, h)


def _head_body(h_ref, wd_ref, bd_ref, gf_ref, bf_ref, wp_ref, bp_ref, o_ref):
    h = h_ref[...]
    a = jnp.maximum(
        jnp.dot(h, wd_ref[...], preferred_element_type=jnp.float32) + bd_ref[...],
        0.0,
    )
    mu = jnp.mean(a, axis=0, keepdims=True)
    var = jnp.mean((a - mu) ** 2, axis=0, keepdims=True)
    hb = (a - mu) * jax.lax.rsqrt(var + 1e-5) * gf_ref[...] + bf_ref[...]
    hg = jnp.tanh(jnp.mean(hb, axis=0, keepdims=True))
    o_ref[...] = jnp.dot(hg, wp_ref[...], preferred_element_type=jnp.float32) + bp_ref[...]


def kernel(x, edge_index, W1, b1, g1, bb1, W2, b2, g2, bb2, Wd, bd, gf, bf, Wp, bp):
    src = edge_index[0]
    dst = edge_index[1]
    deg = jax.ops.segment_sum(jnp.ones((E,), jnp.float32), dst, num_segments=N)

    h = _mean_agg(x, src, dst, deg)
    h = _dense_bn(h, W1, b1, g1, bb1)
    h = _max_agg(h, src, dst, deg)
    h = _mean_agg(h, src, dst, deg)
    h = _dense_bn(h, W2, b2, g2, bb2)
    h = _max_agg(h, src, dst, deg)

    out = pl.pallas_call(
        _head_body,
        out_shape=jax.ShapeDtypeStruct((1, 1), jnp.float32),
    )(h, Wd, bd.reshape(1, H), gf.reshape(1, H), bf.reshape(1, H), Wp, bp.reshape(1, 1))
    return out


# speedup vs baseline: 1.7099x; 1.6674x over previous
import functools

import jax
import jax.numpy as jnp
from jax import lax
from jax.experimental import pallas as pl
from jax.experimental.pallas import tpu as pltpu
from jax.experimental.pallas import tpu_sc as plsc

N = 10000
E = 320000
D = 128
H = 128

NPAD = 10240
NC = 2
NS = 16
NW = NC * NS
EPW = E // NW
CH = 80
NCHUNK = EPW // CH

_mesh = plsc.VectorSubcoreMesh(core_axis_name="c", subcore_axis_name="s")


@functools.partial(
    pl.kernel,
    mesh=_mesh,
    out_type=jax.ShapeDtypeStruct((NC, NPAD, 128), jnp.float32),
    scratch_types=[
        pltpu.VMEM((1, CH), jnp.int32),
        pltpu.VMEM((1, CH), jnp.int32),
        pltpu.VMEM((CH, 128), jnp.float32),
        pltpu.VMEM((80, 128), jnp.float32),
        pltpu.VMEM_SHARED((NPAD, 128), jnp.float32),
        pltpu.SemaphoreType.DMA,
    ],
)
def _mean_deg_sc(h_hbm, src_hbm, dst_hbm, sums_hbm,
                 idx_src, idx_dst, rows, zbuf, acc_sh, sem):
    c = lax.axis_index("c")
    s = lax.axis_index("s")
    wid = c * NS + s

    zero16 = jnp.zeros((16,), jnp.float32)

    @pl.loop(0, 80)
    def _(i):
        for j in range(8):
            zbuf[i, pl.ds(j * 16, 16)] = zero16

    rows_per_tile = NPAD // NS
    base_row = s * rows_per_tile

    @pl.loop(0, rows_per_tile // 80)
    def _(k):
        pltpu.sync_copy(zbuf, acc_sh.at[pl.ds(base_row + k * 80, 80)])

    plsc.subcore_barrier()

    ebase = wid * EPW

    @pl.loop(0, NCHUNK)
    def _(i):
        off = ebase + i * CH
        pltpu.sync_copy(src_hbm.at[pl.ds(off, CH)], idx_src.at[0])
        pltpu.sync_copy(dst_hbm.at[pl.ds(off, CH)], idx_dst.at[0])
        pltpu.async_copy(h_hbm.at[idx_src.at[0]], rows, sem).wait()
        pltpu.sync_copy(rows, acc_sh.at[idx_dst.at[0]], add=True)

    plsc.subcore_barrier()

    @pl.loop(0, rows_per_tile // 80)
    def _(k):
        r = base_row + k * 80
        pltpu.sync_copy(acc_sh.at[pl.ds(r, 80)], sums_hbm.at[c].at[pl.ds(r, 80)])


def _mean_agg_sc(h, src, dst, deg):
    sums2 = _mean_deg_sc(h, src, dst)
    sums = sums2[0, :N] + sums2[1, :N]
    mean = sums / jnp.maximum(deg, 1.0)[:, None]
    return jnp.where((deg > 0)[:, None], mean, h)


def _dense_bn_body(h_ref, w_ref, b_ref, g_ref, bb_ref, o_ref):
    h = h_ref[...]
    a = jnp.maximum(
        jnp.dot(h, w_ref[...], preferred_element_type=jnp.float32) + b_ref[...],
        0.0,
    )
    mu = jnp.mean(a, axis=0, keepdims=True)
    var = jnp.mean((a - mu) ** 2, axis=0, keepdims=True)
    o_ref[...] = (a - mu) * lax.rsqrt(var + 1e-5) * g_ref[...] + bb_ref[...]


def _dense_bn(h, W, b, g, bb):
    return pl.pallas_call(
        _dense_bn_body,
        out_shape=jax.ShapeDtypeStruct((N, H), jnp.float32),
    )(h, W, b.reshape(1, H), g.reshape(1, H), bb.reshape(1, H))


def _max_agg(h, src, dst, deg):
    mx = jax.ops.segment_max(h[src], dst, num_segments=N)
    return jnp.where((deg > 0)[:, None], mx, h)


def _head_body(h_ref, wd_ref, bd_ref, gf_ref, bf_ref, wp_ref, bp_ref, o_ref):
    h = h_ref[...]
    a = jnp.maximum(
        jnp.dot(h, wd_ref[...], preferred_element_type=jnp.float32) + bd_ref[...],
        0.0,
    )
    mu = jnp.mean(a, axis=0, keepdims=True)
    var = jnp.mean((a - mu) ** 2, axis=0, keepdims=True)
    hb = (a - mu) * lax.rsqrt(var + 1e-5) * gf_ref[...] + bf_ref[...]
    hg = jnp.tanh(jnp.mean(hb, axis=0, keepdims=True))
    o_ref[...] = jnp.dot(hg, wp_ref[...], preferred_element_type=jnp.float32) + bp_ref[...]


def kernel(x, edge_index, W1, b1, g1, bb1, W2, b2, g2, bb2, Wd, bd, gf, bf, Wp, bp):
    src = edge_index[0]
    dst = edge_index[1]

    deg = jax.ops.segment_sum(jnp.ones((E,), jnp.float32), dst, num_segments=N)
    h = _mean_agg_sc(x, src, dst, deg)
    h = _dense_bn(h, W1, b1, g1, bb1)
    h = _max_agg(h, src, dst, deg)
    h = _mean_agg_sc(h, src, dst, deg)
    h = _dense_bn(h, W2, b2, g2, bb2)
    h = _max_agg(h, src, dst, deg)

    out = pl.pallas_call(
        _head_body,
        out_shape=jax.ShapeDtypeStruct((1, 1), jnp.float32),
    )(h, Wd, bd.reshape(1, H), gf.reshape(1, H), bf.reshape(1, H), Wp, bp.reshape(1, 1))
    return out
